# baseline (speedup 1.0000x reference)
import functools

import jax
import jax.numpy as jnp
from jax import lax
from jax.experimental import pallas as pl
from jax.experimental.pallas import tpu as pltpu
from jax.experimental.pallas import tpu_sc as plsc

D_MODEL = 1024
SEQ_LEN = 2048
NC, NS, L = 2, 16, 16
NW = NC * NS
B_TOTAL = 4 * SEQ_LEN
B_PER_W = B_TOTAL // NW
CHUNK = 32
N_CHUNKS = B_PER_W // CHUNK
SCALE = 32.0


def _body(seq_hbm, seg_hbm, tok_table, seg_table, pe_hbm, out_hbm,
          idx_v, sidx_v, tok_v, pe_v, seg_v, sem0, sem1, sem2, sem3):
    wid = lax.axis_index("s") * NC + lax.axis_index("c")
    base = wid * B_PER_W
    pos0 = (wid % (SEQ_LEN // B_PER_W)) * B_PER_W

    cp_i = pltpu.async_copy(seq_hbm.at[pl.ds(base, B_PER_W)], idx_v, sem0)
    cp_s = pltpu.async_copy(seg_hbm.at[pl.ds(base, B_PER_W)], sidx_v, sem1)
    cp_i.wait()
    cp_s.wait()

    for c in range(N_CHUNKS):
        off = c * CHUNK
        cp_tok = pltpu.async_copy(
            tok_table.at[idx_v.at[pl.ds(off, CHUNK)]], tok_v, sem0)
        cp_seg = pltpu.async_copy(
            seg_table.at[sidx_v.at[pl.ds(off, CHUNK)]], seg_v, sem1)
        cp_pe = pltpu.async_copy(
            pe_hbm.at[pl.ds(pos0 + off, CHUNK)], pe_v, sem2)
        cp_tok.wait()
        cp_seg.wait()
        cp_pe.wait()

        def row(t, _):
            def vec(j, _):
                sl = pl.ds(j * L, L)
                tok_v[t, sl] = (tok_v[t, sl] * SCALE + pe_v[t, sl]
                                + seg_v[t, sl])
                return 0
            lax.fori_loop(0, D_MODEL // L, vec, 0, unroll=4)
            return 0
        lax.fori_loop(0, CHUNK, row, 0)

        pltpu.async_copy(
            tok_v, out_hbm.at[pl.ds(base + off, CHUNK)], sem3).wait()


@jax.jit
def _embed(seq_flat, seg_flat, tok_table, seg_table, pe):
    mesh = plsc.VectorSubcoreMesh(
        core_axis_name="c", subcore_axis_name="s",
        num_cores=NC, num_subcores=NS)
    return pl.kernel(
        _body,
        out_type=jax.ShapeDtypeStruct((B_TOTAL, D_MODEL), jnp.float32),
        mesh=mesh,
        scratch_types=[
            pltpu.VMEM((B_PER_W,), jnp.int32),
            pltpu.VMEM((B_PER_W,), jnp.int32),
            pltpu.VMEM((CHUNK, D_MODEL), jnp.float32),
            pltpu.VMEM((CHUNK, D_MODEL), jnp.float32),
            pltpu.VMEM((CHUNK, D_MODEL), jnp.float32),
            pltpu.SemaphoreType.DMA,
            pltpu.SemaphoreType.DMA,
            pltpu.SemaphoreType.DMA,
            pltpu.SemaphoreType.DMA,
        ],
    )(seq_flat, seg_flat, tok_table, seg_table, pe)


def kernel(sequence, segment, tok_table, seg_table, pe):
    b, s = sequence.shape
    seq_flat = sequence.reshape(-1).astype(jnp.int32)
    seg_flat = segment.reshape(-1).astype(jnp.int32)
    out = _embed(seq_flat, seg_flat, tok_table, seg_table, pe)
    return out.reshape(b, s, D_MODEL)
